# baseline (speedup 1.0000x reference)
import functools

import jax
import jax.numpy as jnp
from jax import lax
from jax.experimental import pallas as pl
from jax.experimental.pallas import tpu as pltpu
from jax.experimental.pallas import tpu_sc as plsc

_N = 10000
_E = 320000
_D = 128
_NC = 2
_NS = 16
_NW = _NC * _NS
_CHUNK = 128
_NCH = -(-_E // (_NW * _CHUNK))
_EPAD = _NW * _NCH * _CHUNK
_NPAD = 10240
_RPT = _NPAD // _NS
_NIB = 4


def _spmm_sc(x, src3, dst3, w3, with_rs):
    mesh = plsc.VectorSubcoreMesh(core_axis_name="c", subcore_axis_name="s")

    acc_t = jax.ShapeDtypeStruct((_NC, _NPAD, _D), jnp.float32)
    out_type = ((acc_t, jax.ShapeDtypeStruct((_NC, _NPAD), jnp.float32))
                if with_rs else acc_t)

    @functools.partial(
        pl.kernel,
        mesh=mesh,
        out_type=out_type,
        scratch_types=[
            pltpu.VMEM((_NIB, _CHUNK), jnp.int32),
            pltpu.VMEM((_NIB, _CHUNK), jnp.int32),
            pltpu.VMEM((_NIB, _CHUNK), jnp.float32),
            pltpu.VMEM((2, _CHUNK, _D), jnp.float32),
            pltpu.VMEM_SHARED((_NPAD, _D), jnp.float32),
            pltpu.VMEM_SHARED((_NPAD,), jnp.float32),
            pltpu.SemaphoreType.DMA,
            pltpu.SemaphoreType.DMA,
            pltpu.SemaphoreType.DMA,
        ],
    )
    def spmm(x_hbm, src_hbm, dst_hbm, w_hbm, out_hbm, *rest):
        if with_rs:
            rs_hbm = rest[0]
            rest = rest[1:]
        (src_b, dst_b, w_b, gbuf, acc_sh, rs_sh,
         sem_g, sem_i, sem_s) = rest
        c = lax.axis_index("c")
        s = lax.axis_index("s")
        wid = c * _NS + s
        base = s * _RPT

        def _zrow(i, _):
            for k in range(_D // 16):
                gbuf[0, i, pl.ds(16 * k, 16)] = jnp.zeros((16,), jnp.float32)
            return 0
        lax.fori_loop(0, _CHUNK, _zrow, 0)
        for j in range(_RPT // _CHUNK):
            pltpu.async_copy(gbuf.at[0],
                             acc_sh.at[pl.ds(base + j * _CHUNK, _CHUNK)],
                             sem_s)
            if with_rs:
                pltpu.async_copy(gbuf.at[0, 0, pl.ds(0, _CHUNK)],
                                 rs_sh.at[pl.ds(base + j * _CHUNK, _CHUNK)],
                                 sem_s)
        for j in range(_RPT // _CHUNK):
            pltpu.make_async_copy(
                gbuf.at[0],
                acc_sh.at[pl.ds(base + j * _CHUNK, _CHUNK)], sem_s).wait()
            if with_rs:
                pltpu.make_async_copy(
                    gbuf.at[0, 0, pl.ds(0, _CHUNK)],
                    rs_sh.at[pl.ds(base + j * _CHUNK, _CHUNK)], sem_s).wait()

        def _fire_idx(ci):
            bi = lax.rem(ci, _NIB)
            pltpu.async_copy(src_hbm.at[wid, ci], src_b.at[bi], sem_i)
            pltpu.async_copy(dst_hbm.at[wid, ci], dst_b.at[bi], sem_i)
            pltpu.async_copy(w_hbm.at[wid, ci], w_b.at[bi], sem_i)

        def _wait_idx(ci):
            bi = lax.rem(ci, _NIB)
            pltpu.make_async_copy(src_hbm.at[wid, ci], src_b.at[bi], sem_i).wait()
            pltpu.make_async_copy(dst_hbm.at[wid, ci], dst_b.at[bi], sem_i).wait()
            pltpu.make_async_copy(w_hbm.at[wid, ci], w_b.at[bi], sem_i).wait()

        def _fire_gather(ci):
            gi = lax.rem(ci, 2)
            pltpu.async_copy(x_hbm.at[src_b.at[lax.rem(ci, _NIB)]],
                             gbuf.at[gi], sem_g)

        def _wait_gather(ci):
            gi = lax.rem(ci, 2)
            pltpu.make_async_copy(x_hbm.at[src_b.at[lax.rem(ci, _NIB)]],
                                  gbuf.at[gi], sem_g).wait()

        def _fire_scatter(ci):
            bi = lax.rem(ci, _NIB)
            si = lax.rem(ci, 2)
            pltpu.async_copy(gbuf.at[si], acc_sh.at[dst_b.at[bi]], sem_s,
                             add=True)
            if with_rs:
                pltpu.async_copy(w_b.at[bi], rs_sh.at[dst_b.at[bi]], sem_s,
                                 add=True)

        def _wait_scatter(ci):
            bi = lax.rem(ci, _NIB)
            si = lax.rem(ci, 2)
            pltpu.make_async_copy(gbuf.at[si], acc_sh.at[dst_b.at[bi]],
                                  sem_s).wait()
            if with_rs:
                pltpu.make_async_copy(w_b.at[bi], rs_sh.at[dst_b.at[bi]],
                                      sem_s).wait()

        plsc.subcore_barrier()

        _fire_idx(0)
        _wait_idx(0)
        _fire_gather(0)
        _fire_idx(1)

        def chunk_body(ci, _):
            @pl.when(ci >= 1)
            def _():
                _wait_scatter(ci - 1)

            _wait_gather(ci)

            @pl.when(ci + 1 < _NCH)
            def _():
                _wait_idx(ci + 1)
                _fire_gather(ci + 1)

            @pl.when(ci + 2 < _NCH)
            def _():
                _fire_idx(ci + 2)

            bi = lax.rem(ci, _NIB)
            si = lax.rem(ci, 2)

            @plsc.parallel_loop(0, _CHUNK // 16, unroll=2)
            def grp_body(g_):
                wvec = w_b[bi, pl.ds(16 * g_, 16)]
                for j in range(16):
                    e = 16 * g_ + j
                    wb = lax.gather(
                        wvec, jnp.full((16, 1), j, jnp.int32),
                        lax.GatherDimensionNumbers(
                            offset_dims=(), collapsed_slice_dims=(0,),
                            start_index_map=(0,)),
                        slice_sizes=(1,),
                        mode=lax.GatherScatterMode.PROMISE_IN_BOUNDS)
                    for k in range(_D // 16):
                        gbuf[si, e, pl.ds(16 * k, 16)] = (
                            gbuf[si, e, pl.ds(16 * k, 16)] * wb)

            _fire_scatter(ci)
            return 0
        lax.fori_loop(0, _NCH, chunk_body, 0)

        _wait_scatter(_NCH - 1)

        plsc.subcore_barrier()

        pltpu.sync_copy(acc_sh.at[pl.ds(base, _RPT)],
                        out_hbm.at[c, pl.ds(base, _RPT)])
        if with_rs:
            pltpu.sync_copy(rs_sh.at[pl.ds(base, _RPT)],
                            rs_hbm.at[c, pl.ds(base, _RPT)])

    if with_rs:
        acc, rs = spmm(x, src3, dst3, w3)
        return acc, rs.reshape(_NC, _NPAD, 1)
    return spmm(x, src3, dst3, w3)


def _mlp_body(x_ref, p_ref, rs_ref, w_ref, b_ref, o_ref, *, relu):
    acc = p_ref[0] + p_ref[1]
    rs = rs_ref[0] + rs_ref[1]
    nacc = acc / jnp.maximum(rs, 1e-12)
    cat = jnp.concatenate([x_ref[...], nacc], axis=1)
    h = lax.dot_general(cat, w_ref[...], (((1,), (1,)), ((), ())),
                        preferred_element_type=jnp.float32) + b_ref[...]
    o_ref[...] = jnp.maximum(h, 0.0) if relu else h


_BLK = 1000


def _mlp_tc(x, part, rs, W, b, relu):
    body = functools.partial(_mlp_body, relu=relu)
    return pl.pallas_call(
        body,
        grid=(_N // _BLK,),
        in_specs=[
            pl.BlockSpec((_BLK, _D), lambda i: (i, 0)),
            pl.BlockSpec((_NC, _BLK, _D), lambda i: (0, i, 0)),
            pl.BlockSpec((_NC, _BLK, 1), lambda i: (0, i, 0)),
            pl.BlockSpec(W.shape, lambda i: (0, 0)),
            pl.BlockSpec((1, _D), lambda i: (0, 0)),
        ],
        out_specs=pl.BlockSpec((_BLK, _D), lambda i: (i, 0)),
        out_shape=jax.ShapeDtypeStruct((_N, _D), jnp.float32),
    )(x, part, rs, W, b)


def kernel(x, edge_index, edge_weight, W1_w, W1_b, W2_w, W2_b):
    dst = edge_index[0]
    src = edge_index[1]
    pad = _EPAD - _E
    src3 = jnp.concatenate([src, jnp.zeros((pad,), src.dtype)]).reshape(
        _NW, _NCH, _CHUNK)
    dst3 = jnp.concatenate([dst, jnp.zeros((pad,), dst.dtype)]).reshape(
        _NW, _NCH, _CHUNK)
    w3 = jnp.concatenate(
        [edge_weight, jnp.zeros((pad,), edge_weight.dtype)]).reshape(
        _NW, _NCH, _CHUNK)

    part1, rs1 = _spmm_sc(x, src3, dst3, w3, with_rs=True)
    h = _mlp_tc(x, part1, rs1, W1_w, W1_b.reshape(1, _D), relu=True)
    part2 = _spmm_sc(h, src3, dst3, w3, with_rs=False)
    return _mlp_tc(h, part2, rs1, W2_w, W2_b.reshape(1, _D), relu=False)


# speedup vs baseline: 1.5607x; 1.5607x over previous
import functools

import jax
import jax.numpy as jnp
from jax import lax
from jax.experimental import pallas as pl
from jax.experimental.pallas import tpu as pltpu
from jax.experimental.pallas import tpu_sc as plsc

_N = 10000
_E = 320000
_D = 128
_NC = 2
_NS = 16
_NW = _NC * _NS
_CHUNK = 64
_NCH = -(-_E // (_NW * _CHUNK))
_EPAD = _NW * _NCH * _CHUNK
_NPAD = 10240
_RPT = _NPAD // _NS
_NIB = 4
_NSB = 2


def _spmm_sc(x, src3, dst3, w3, with_rs):
    mesh = plsc.VectorSubcoreMesh(core_axis_name="c", subcore_axis_name="s")

    acc_t = jax.ShapeDtypeStruct((_NC, _NPAD, _D), jnp.float32)
    out_type = ((acc_t, jax.ShapeDtypeStruct((_NC, _NPAD), jnp.float32))
                if with_rs else acc_t)

    @functools.partial(
        pl.kernel,
        mesh=mesh,
        out_type=out_type,
        scratch_types=[
            pltpu.VMEM((_NIB, _CHUNK), jnp.int32),
            pltpu.VMEM((_NIB, _CHUNK), jnp.int32),
            pltpu.VMEM((_NIB, _CHUNK), jnp.float32),
            pltpu.VMEM((2, _CHUNK, _D), jnp.float32),
            pltpu.VMEM((_NSB, _CHUNK, _D), jnp.float32),
            pltpu.VMEM_SHARED((_NPAD, _D), jnp.float32),
            pltpu.VMEM_SHARED((_NPAD,), jnp.float32),
            pltpu.SemaphoreType.DMA,
            pltpu.SemaphoreType.DMA,
            pltpu.SemaphoreType.DMA,
        ],
    )
    def spmm(x_hbm, src_hbm, dst_hbm, w_hbm, out_hbm, *rest):
        if with_rs:
            rs_hbm = rest[0]
            rest = rest[1:]
        (src_b, dst_b, w_b, gbuf, sbuf, acc_sh, rs_sh,
         sem_g, sem_i, sem_s) = rest
        c = lax.axis_index("c")
        s = lax.axis_index("s")
        wid = c * _NS + s
        base = s * _RPT

        def _zrow(i, _):
            for k in range(_D // 16):
                sbuf[0, i, pl.ds(16 * k, 16)] = jnp.zeros((16,), jnp.float32)
            return 0
        lax.fori_loop(0, _CHUNK, _zrow, 0)
        for j in range(_RPT // _CHUNK):
            pltpu.async_copy(sbuf.at[0],
                             acc_sh.at[pl.ds(base + j * _CHUNK, _CHUNK)],
                             sem_s)
            if with_rs:
                pltpu.async_copy(sbuf.at[0, 0, pl.ds(0, _CHUNK)],
                                 rs_sh.at[pl.ds(base + j * _CHUNK, _CHUNK)],
                                 sem_s)
        for j in range(_RPT // _CHUNK):
            pltpu.make_async_copy(
                sbuf.at[0],
                acc_sh.at[pl.ds(base + j * _CHUNK, _CHUNK)], sem_s).wait()
            if with_rs:
                pltpu.make_async_copy(
                    sbuf.at[0, 0, pl.ds(0, _CHUNK)],
                    rs_sh.at[pl.ds(base + j * _CHUNK, _CHUNK)], sem_s).wait()

        def _fire_idx(ci):
            bi = lax.rem(ci, _NIB)
            pltpu.async_copy(src_hbm.at[wid, ci], src_b.at[bi], sem_i)
            pltpu.async_copy(dst_hbm.at[wid, ci], dst_b.at[bi], sem_i)
            pltpu.async_copy(w_hbm.at[wid, ci], w_b.at[bi], sem_i)

        def _wait_idx(ci):
            bi = lax.rem(ci, _NIB)
            pltpu.make_async_copy(src_hbm.at[wid, ci], src_b.at[bi], sem_i).wait()
            pltpu.make_async_copy(dst_hbm.at[wid, ci], dst_b.at[bi], sem_i).wait()
            pltpu.make_async_copy(w_hbm.at[wid, ci], w_b.at[bi], sem_i).wait()

        def _fire_gather(ci):
            gi = lax.rem(ci, 2)
            pltpu.async_copy(x_hbm.at[src_b.at[lax.rem(ci, _NIB)]],
                             gbuf.at[gi], sem_g)

        def _wait_gather(ci):
            gi = lax.rem(ci, 2)
            pltpu.make_async_copy(x_hbm.at[src_b.at[lax.rem(ci, _NIB)]],
                                  gbuf.at[gi], sem_g).wait()

        def _fire_scatter(ci):
            bi = lax.rem(ci, _NIB)
            si = lax.rem(ci, _NSB)
            pltpu.async_copy(sbuf.at[si], acc_sh.at[dst_b.at[bi]], sem_s,
                             add=True)
            if with_rs:
                pltpu.async_copy(w_b.at[bi], rs_sh.at[dst_b.at[bi]], sem_s,
                                 add=True)

        def _wait_scatter(ci):
            bi = lax.rem(ci, _NIB)
            si = lax.rem(ci, _NSB)
            pltpu.make_async_copy(sbuf.at[si], acc_sh.at[dst_b.at[bi]],
                                  sem_s).wait()
            if with_rs:
                pltpu.make_async_copy(w_b.at[bi], rs_sh.at[dst_b.at[bi]],
                                      sem_s).wait()

        plsc.subcore_barrier()

        _fire_idx(0)
        _wait_idx(0)
        _fire_gather(0)
        _fire_idx(1)

        def chunk_body(ci, _):
            @pl.when(ci >= _NSB)
            def _():
                _wait_scatter(ci - _NSB)

            _wait_gather(ci)

            @pl.when(ci + 1 < _NCH)
            def _():
                _wait_idx(ci + 1)
                _fire_gather(ci + 1)

            @pl.when(ci + 2 < _NCH)
            def _():
                _fire_idx(ci + 2)

            bi = lax.rem(ci, _NIB)
            si = lax.rem(ci, _NSB)

            @plsc.parallel_loop(0, _CHUNK // 16, unroll=2)
            def grp_body(g_):
                wvec = w_b[bi, pl.ds(16 * g_, 16)]
                for j in range(16):
                    e = 16 * g_ + j
                    wb = lax.gather(
                        wvec, jnp.full((16, 1), j, jnp.int32),
                        lax.GatherDimensionNumbers(
                            offset_dims=(), collapsed_slice_dims=(0,),
                            start_index_map=(0,)),
                        slice_sizes=(1,),
                        mode=lax.GatherScatterMode.PROMISE_IN_BOUNDS)
                    for k in range(_D // 16):
                        sbuf[si, e, pl.ds(16 * k, 16)] = (
                            gbuf[si, e, pl.ds(16 * k, 16)] * wb)

            _fire_scatter(ci)
            return 0
        lax.fori_loop(0, _NCH, chunk_body, 0)

        for t in range(min(_NSB, _NCH)):
            _wait_scatter(_NCH - min(_NSB, _NCH) + t)

        plsc.subcore_barrier()

        pltpu.sync_copy(acc_sh.at[pl.ds(base, _RPT)],
                        out_hbm.at[c, pl.ds(base, _RPT)])
        if with_rs:
            pltpu.sync_copy(rs_sh.at[pl.ds(base, _RPT)],
                            rs_hbm.at[c, pl.ds(base, _RPT)])

    if with_rs:
        acc, rs = spmm(x, src3, dst3, w3)
        return acc, rs.reshape(_NC, _NPAD, 1)
    return spmm(x, src3, dst3, w3)


def _mlp_body(x_ref, p_ref, rs_ref, w_ref, b_ref, o_ref, *, relu):
    acc = p_ref[0] + p_ref[1]
    rs = rs_ref[0] + rs_ref[1]
    nacc = acc / jnp.maximum(rs, 1e-12)
    cat = jnp.concatenate([x_ref[...], nacc], axis=1)
    h = lax.dot_general(cat, w_ref[...], (((1,), (1,)), ((), ())),
                        preferred_element_type=jnp.float32) + b_ref[...]
    o_ref[...] = jnp.maximum(h, 0.0) if relu else h


_BLK = 1000


def _mlp_tc(x, part, rs, W, b, relu):
    body = functools.partial(_mlp_body, relu=relu)
    return pl.pallas_call(
        body,
        grid=(_N // _BLK,),
        in_specs=[
            pl.BlockSpec((_BLK, _D), lambda i: (i, 0)),
            pl.BlockSpec((_NC, _BLK, _D), lambda i: (0, i, 0)),
            pl.BlockSpec((_NC, _BLK, 1), lambda i: (0, i, 0)),
            pl.BlockSpec(W.shape, lambda i: (0, 0)),
            pl.BlockSpec((1, _D), lambda i: (0, 0)),
        ],
        out_specs=pl.BlockSpec((_BLK, _D), lambda i: (i, 0)),
        out_shape=jax.ShapeDtypeStruct((_N, _D), jnp.float32),
    )(x, part, rs, W, b)


def kernel(x, edge_index, edge_weight, W1_w, W1_b, W2_w, W2_b):
    dst = edge_index[0]
    src = edge_index[1]
    pad = _EPAD - _E
    src3 = jnp.concatenate([src, jnp.zeros((pad,), src.dtype)]).reshape(
        _NW, _NCH, _CHUNK)
    dst3 = jnp.concatenate([dst, jnp.zeros((pad,), dst.dtype)]).reshape(
        _NW, _NCH, _CHUNK)
    w3 = jnp.concatenate(
        [edge_weight, jnp.zeros((pad,), edge_weight.dtype)]).reshape(
        _NW, _NCH, _CHUNK)

    part1, rs1 = _spmm_sc(x, src3, dst3, w3, with_rs=True)
    h = _mlp_tc(x, part1, rs1, W1_w, W1_b.reshape(1, _D), relu=True)
    part2 = _spmm_sc(h, src3, dst3, w3, with_rs=False)
    return _mlp_tc(h, part2, rs1, W2_w, W2_b.reshape(1, _D), relu=False)


# speedup vs baseline: 1.6851x; 1.0797x over previous
import functools

import jax
import jax.numpy as jnp
from jax import lax
from jax.experimental import pallas as pl
from jax.experimental.pallas import tpu as pltpu
from jax.experimental.pallas import tpu_sc as plsc

_N = 10000
_E = 320000
_D = 128
_NC = 2
_NS = 16
_NW = _NC * _NS
_CHUNK = 64
_NCH = -(-_E // (_NW * _CHUNK))
_EPAD = _NW * _NCH * _CHUNK
_NPAD = 10240
_RPT = _NPAD // _NS
_NIB = 4
_NSB = 2


_PERM = tuple(
    32 * k + (j // 2 if j % 2 == 0 else 16 + j // 2)
    for k in range(_D // 32) for j in range(32))


def _spmm_sc(x, src3, dst3, w3, with_rs):
    mesh = plsc.VectorSubcoreMesh(core_axis_name="c", subcore_axis_name="s")

    acc_t = jax.ShapeDtypeStruct((_NC, _NPAD, _D), jnp.float32)
    out_type = ((acc_t, jax.ShapeDtypeStruct((_NC, _NPAD), jnp.float32))
                if with_rs else acc_t)

    @functools.partial(
        pl.kernel,
        mesh=mesh,
        out_type=out_type,
        compiler_params=pltpu.CompilerParams(needs_layout_passes=False, use_tc_tiling_on_sc=False),
        scratch_types=[
            pltpu.VMEM((_NIB, _CHUNK), jnp.int32),
            pltpu.VMEM((_NIB, _CHUNK), jnp.int32),
            pltpu.VMEM((_NIB, _CHUNK), jnp.float32),
            pltpu.VMEM((2, _CHUNK, _D // 2), jnp.int32),
            pltpu.VMEM((_NSB, _CHUNK, _D), jnp.float32),
            pltpu.VMEM_SHARED((_NPAD, _D), jnp.float32),
            pltpu.VMEM_SHARED((_NPAD,), jnp.float32),
            pltpu.SemaphoreType.DMA,
            pltpu.SemaphoreType.DMA,
            pltpu.SemaphoreType.DMA,
        ],
    )
    def spmm(x_hbm, src_hbm, dst_hbm, w_hbm, out_hbm, *rest):
        if with_rs:
            rs_hbm = rest[0]
            rest = rest[1:]
        (src_b, dst_b, w_b, gbuf, sbuf, acc_sh, rs_sh,
         sem_g, sem_i, sem_s) = rest
        c = lax.axis_index("c")
        s = lax.axis_index("s")
        wid = c * _NS + s
        base = s * _RPT

        def _zrow(i, _):
            for k in range(_D // 16):
                sbuf[0, i, pl.ds(16 * k, 16)] = jnp.zeros((16,), jnp.float32)
            return 0
        lax.fori_loop(0, _CHUNK, _zrow, 0)
        for j in range(_RPT // _CHUNK):
            pltpu.async_copy(sbuf.at[0],
                             acc_sh.at[pl.ds(base + j * _CHUNK, _CHUNK)],
                             sem_s)
            if with_rs:
                pltpu.async_copy(sbuf.at[0, 0, pl.ds(0, _CHUNK)],
                                 rs_sh.at[pl.ds(base + j * _CHUNK, _CHUNK)],
                                 sem_s)
        for j in range(_RPT // _CHUNK):
            pltpu.make_async_copy(
                sbuf.at[0],
                acc_sh.at[pl.ds(base + j * _CHUNK, _CHUNK)], sem_s).wait()
            if with_rs:
                pltpu.make_async_copy(
                    sbuf.at[0, 0, pl.ds(0, _CHUNK)],
                    rs_sh.at[pl.ds(base + j * _CHUNK, _CHUNK)], sem_s).wait()

        def _fire_idx(ci):
            bi = lax.rem(ci, _NIB)
            pltpu.async_copy(src_hbm.at[wid, ci], src_b.at[bi], sem_i)
            pltpu.async_copy(dst_hbm.at[wid, ci], dst_b.at[bi], sem_i)
            pltpu.async_copy(w_hbm.at[wid, ci], w_b.at[bi], sem_i)

        def _wait_idx(ci):
            bi = lax.rem(ci, _NIB)
            pltpu.make_async_copy(src_hbm.at[wid, ci], src_b.at[bi], sem_i).wait()
            pltpu.make_async_copy(dst_hbm.at[wid, ci], dst_b.at[bi], sem_i).wait()
            pltpu.make_async_copy(w_hbm.at[wid, ci], w_b.at[bi], sem_i).wait()

        def _fire_gather(ci):
            gi = lax.rem(ci, 2)
            pltpu.async_copy(x_hbm.at[src_b.at[lax.rem(ci, _NIB)]],
                             gbuf.at[gi], sem_g)

        def _wait_gather(ci):
            gi = lax.rem(ci, 2)
            pltpu.make_async_copy(x_hbm.at[src_b.at[lax.rem(ci, _NIB)]],
                                  gbuf.at[gi], sem_g).wait()

        def _fire_scatter(ci):
            bi = lax.rem(ci, _NIB)
            si = lax.rem(ci, _NSB)
            pltpu.async_copy(sbuf.at[si], acc_sh.at[dst_b.at[bi]], sem_s,
                             add=True)
            if with_rs:
                pltpu.async_copy(w_b.at[bi], rs_sh.at[dst_b.at[bi]], sem_s,
                                 add=True)

        def _wait_scatter(ci):
            bi = lax.rem(ci, _NIB)
            si = lax.rem(ci, _NSB)
            pltpu.make_async_copy(sbuf.at[si], acc_sh.at[dst_b.at[bi]],
                                  sem_s).wait()
            if with_rs:
                pltpu.make_async_copy(w_b.at[bi], rs_sh.at[dst_b.at[bi]],
                                      sem_s).wait()

        plsc.subcore_barrier()

        _fire_idx(0)
        _wait_idx(0)
        _fire_gather(0)
        _fire_idx(1)

        def chunk_body(ci, _):
            @pl.when(ci >= _NSB)
            def _():
                _wait_scatter(ci - _NSB)

            _wait_gather(ci)

            @pl.when(ci + 1 < _NCH)
            def _():
                _wait_idx(ci + 1)
                _fire_gather(ci + 1)

            @pl.when(ci + 2 < _NCH)
            def _():
                _fire_idx(ci + 2)

            bi = lax.rem(ci, _NIB)
            si = lax.rem(ci, _NSB)

            @plsc.parallel_loop(0, _CHUNK // 16, unroll=2)
            def grp_body(g_):
                wvec = w_b[bi, pl.ds(16 * g_, 16)]
                for j in range(16):
                    e = 16 * g_ + j
                    wb = lax.gather(
                        wvec, jnp.full((16, 1), j, jnp.int32),
                        lax.GatherDimensionNumbers(
                            offset_dims=(), collapsed_slice_dims=(0,),
                            start_index_map=(0,)),
                        slice_sizes=(1,),
                        mode=lax.GatherScatterMode.PROMISE_IN_BOUNDS)
                    for k in range(_D // 32):
                        vi = gbuf[si, e, pl.ds(16 * k, 16)]
                        a = plsc.bitcast(
                            lax.shift_left(vi, 16), jnp.float32)
                        b = plsc.bitcast(
                            jnp.bitwise_and(vi, jnp.int32(-65536)),
                            jnp.float32)
                        sbuf[si, e, pl.ds(32 * k, 16)] = a * wb
                        sbuf[si, e, pl.ds(32 * k + 16, 16)] = b * wb

            _fire_scatter(ci)
            return 0
        lax.fori_loop(0, _NCH, chunk_body, 0)

        for t in range(min(_NSB, _NCH)):
            _wait_scatter(_NCH - min(_NSB, _NCH) + t)

        plsc.subcore_barrier()

        pltpu.sync_copy(acc_sh.at[pl.ds(base, _RPT)],
                        out_hbm.at[c, pl.ds(base, _RPT)])
        if with_rs:
            pltpu.sync_copy(rs_sh.at[pl.ds(base, _RPT)],
                            rs_hbm.at[c, pl.ds(base, _RPT)])

    if with_rs:
        acc, rs = spmm(x, src3, dst3, w3)
        return acc, rs.reshape(_NC, _NPAD, 1)
    return spmm(x, src3, dst3, w3)


def _mlp_body(x_ref, p_ref, rs_ref, w_ref, b_ref, o_ref, *, relu):
    acc = p_ref[0] + p_ref[1]
    rs = rs_ref[0] + rs_ref[1]
    nacc = acc / jnp.maximum(rs, 1e-12)
    cat = jnp.concatenate([x_ref[...], nacc], axis=1)
    h = lax.dot_general(cat, w_ref[...], (((1,), (1,)), ((), ())),
                        preferred_element_type=jnp.float32) + b_ref[...]
    o_ref[...] = jnp.maximum(h, 0.0) if relu else h


_BLK = 1000


def _mlp_tc(x, part, rs, W, b, relu):
    body = functools.partial(_mlp_body, relu=relu)
    return pl.pallas_call(
        body,
        grid=(_N // _BLK,),
        in_specs=[
            pl.BlockSpec((_BLK, _D), lambda i: (i, 0)),
            pl.BlockSpec((_NC, _BLK, _D), lambda i: (0, i, 0)),
            pl.BlockSpec((_NC, _BLK, 1), lambda i: (0, i, 0)),
            pl.BlockSpec(W.shape, lambda i: (0, 0)),
            pl.BlockSpec((1, _D), lambda i: (0, 0)),
        ],
        out_specs=pl.BlockSpec((_BLK, _D), lambda i: (i, 0)),
        out_shape=jax.ShapeDtypeStruct((_N, _D), jnp.float32),
    )(x, part, rs, W, b)


def kernel(x, edge_index, edge_weight, W1_w, W1_b, W2_w, W2_b):
    dst = edge_index[0]
    src = edge_index[1]
    pad = _EPAD - _E
    src3 = jnp.concatenate([src, jnp.zeros((pad,), src.dtype)]).reshape(
        _NW, _NCH, _CHUNK)
    dst3 = jnp.concatenate([dst, jnp.zeros((pad,), dst.dtype)]).reshape(
        _NW, _NCH, _CHUNK)
    w3 = jnp.concatenate(
        [edge_weight, jnp.zeros((pad,), edge_weight.dtype)]).reshape(
        _NW, _NCH, _CHUNK)

    perm = jnp.array(_PERM, dtype=jnp.int32)

    def _pack(a):
        ab = jnp.take(a.astype(jnp.bfloat16), perm, axis=1)
        return lax.bitcast_convert_type(
            ab.reshape(_N, _D // 2, 2), jnp.int32)

    part1, rs1 = _spmm_sc(_pack(x), src3, dst3, w3, with_rs=True)
    h = _mlp_tc(x, part1, rs1, W1_w, W1_b.reshape(1, _D), relu=True)
    part2 = _spmm_sc(_pack(h), src3, dst3, w3, with_rs=False)
    return _mlp_tc(h, part2, rs1, W2_w, W2_b.reshape(1, _D), relu=False)


# speedup vs baseline: 1.7514x; 1.0394x over previous
import functools

import jax
import jax.numpy as jnp
from jax import lax
from jax.experimental import pallas as pl
from jax.experimental.pallas import tpu as pltpu
from jax.experimental.pallas import tpu_sc as plsc

_N = 10000
_E = 320000
_D = 128
_NC = 2
_NS = 16
_NW = _NC * _NS
_CHUNK = 64
_NCH = -(-_E // (_NW * _CHUNK))
_EPAD = _NW * _NCH * _CHUNK
_NPAD = 10240
_RPT = _NPAD // _NS
_NIB = 8
_NSB = 3
_NGB = 3


_PERM = tuple(
    32 * k + (j // 2 if j % 2 == 0 else 16 + j // 2)
    for k in range(_D // 32) for j in range(32))


def _spmm_sc(x, src3, dst3, w3, with_rs):
    mesh = plsc.VectorSubcoreMesh(core_axis_name="c", subcore_axis_name="s")

    acc_t = jax.ShapeDtypeStruct((_NC, _NPAD, _D), jnp.float32)
    out_type = ((acc_t, jax.ShapeDtypeStruct((_NC, _NPAD), jnp.float32))
                if with_rs else acc_t)

    @functools.partial(
        pl.kernel,
        mesh=mesh,
        out_type=out_type,
        compiler_params=pltpu.CompilerParams(needs_layout_passes=False, use_tc_tiling_on_sc=False),
        scratch_types=[
            pltpu.VMEM((_NIB, _CHUNK), jnp.int32),
            pltpu.VMEM((_NIB, _CHUNK), jnp.int32),
            pltpu.VMEM((_NIB, _CHUNK), jnp.float32),
            pltpu.VMEM((_NGB, _CHUNK, _D // 2), jnp.int32),
            pltpu.VMEM((_NSB, _CHUNK, _D), jnp.float32),
            pltpu.VMEM_SHARED((_NPAD, _D), jnp.float32),
            pltpu.VMEM_SHARED((_NPAD,), jnp.float32),
            pltpu.SemaphoreType.DMA,
            pltpu.SemaphoreType.DMA,
            pltpu.SemaphoreType.DMA,
            pltpu.SemaphoreType.DMA,
            pltpu.SemaphoreType.DMA,
            pltpu.SemaphoreType.DMA,
            pltpu.SemaphoreType.DMA,
        ],
    )
    def spmm(x_hbm, src_hbm, dst_hbm, w_hbm, out_hbm, *rest):
        if with_rs:
            rs_hbm = rest[0]
            rest = rest[1:]
        (src_b, dst_b, w_b, gbuf, sbuf, acc_sh, rs_sh,
         sem_g0, sem_g1, sem_g2, sem_i, sem_s0, sem_s1, sem_s2) = rest
        sem_gs = (sem_g0, sem_g1, sem_g2)
        sem_ss = (sem_s0, sem_s1, sem_s2)
        c = lax.axis_index("c")
        s = lax.axis_index("s")
        wid = c * _NS + s
        base = s * _RPT

        def _zrow(i, _):
            for k in range(_D // 16):
                sbuf[0, i, pl.ds(16 * k, 16)] = jnp.zeros((16,), jnp.float32)
            return 0
        lax.fori_loop(0, _CHUNK, _zrow, 0)
        for j in range(_RPT // _CHUNK):
            pltpu.async_copy(sbuf.at[0],
                             acc_sh.at[pl.ds(base + j * _CHUNK, _CHUNK)],
                             sem_s0)
            if with_rs:
                pltpu.async_copy(sbuf.at[0, 0, pl.ds(0, _CHUNK)],
                                 rs_sh.at[pl.ds(base + j * _CHUNK, _CHUNK)],
                                 sem_s0)
        for j in range(_RPT // _CHUNK):
            pltpu.make_async_copy(
                sbuf.at[0],
                acc_sh.at[pl.ds(base + j * _CHUNK, _CHUNK)], sem_s0).wait()
            if with_rs:
                pltpu.make_async_copy(
                    sbuf.at[0, 0, pl.ds(0, _CHUNK)],
                    rs_sh.at[pl.ds(base + j * _CHUNK, _CHUNK)], sem_s0).wait()

        def _fire_idx(ci):
            bi = lax.rem(ci, _NIB)
            pltpu.async_copy(src_hbm.at[wid, ci], src_b.at[bi], sem_i)
            pltpu.async_copy(dst_hbm.at[wid, ci], dst_b.at[bi], sem_i)
            pltpu.async_copy(w_hbm.at[wid, ci], w_b.at[bi], sem_i)

        def _wait_idx(ci):
            bi = lax.rem(ci, _NIB)
            pltpu.make_async_copy(src_hbm.at[wid, ci], src_b.at[bi], sem_i).wait()
            pltpu.make_async_copy(dst_hbm.at[wid, ci], dst_b.at[bi], sem_i).wait()
            pltpu.make_async_copy(w_hbm.at[wid, ci], w_b.at[bi], sem_i).wait()

        def _fire_gather(ci):
            gi = lax.rem(ci, _NGB)
            for k in range(_NGB):
                @pl.when(gi == k)
                def _(k=k):
                    pltpu.async_copy(x_hbm.at[src_b.at[lax.rem(ci, _NIB)]],
                                     gbuf.at[k], sem_gs[k])

        def _wait_gather(ci):
            gi = lax.rem(ci, _NGB)
            for k in range(_NGB):
                @pl.when(gi == k)
                def _(k=k):
                    pltpu.make_async_copy(
                        x_hbm.at[src_b.at[lax.rem(ci, _NIB)]],
                        gbuf.at[k], sem_gs[k]).wait()

        def _fire_scatter(ci):
            bi = lax.rem(ci, _NIB)
            si = lax.rem(ci, _NSB)
            for k in range(_NSB):
                @pl.when(si == k)
                def _(k=k):
                    pltpu.async_copy(sbuf.at[k], acc_sh.at[dst_b.at[bi]],
                                     sem_ss[k], add=True)
                    if with_rs:
                        pltpu.async_copy(w_b.at[bi], rs_sh.at[dst_b.at[bi]],
                                         sem_ss[k], add=True)

        def _wait_scatter(ci):
            bi = lax.rem(ci, _NIB)
            si = lax.rem(ci, _NSB)
            for k in range(_NSB):
                @pl.when(si == k)
                def _(k=k):
                    pltpu.make_async_copy(sbuf.at[k], acc_sh.at[dst_b.at[bi]],
                                          sem_ss[k]).wait()
                    if with_rs:
                        pltpu.make_async_copy(w_b.at[bi],
                                              rs_sh.at[dst_b.at[bi]],
                                              sem_ss[k]).wait()

        plsc.subcore_barrier()

        _fire_idx(0)
        _fire_idx(1)
        _fire_idx(2)
        _wait_idx(0)
        _fire_gather(0)
        _wait_idx(1)
        _fire_gather(1)

        def chunk_body(ci, _):
            @pl.when(ci >= _NSB)
            def _():
                _wait_scatter(ci - _NSB)

            _wait_gather(ci)

            @pl.when(ci + 2 < _NCH)
            def _():
                _wait_idx(ci + 2)
                _fire_gather(ci + 2)

            @pl.when(ci + 3 < _NCH)
            def _():
                _fire_idx(ci + 3)

            bi = lax.rem(ci, _NIB)
            si = lax.rem(ci, _NSB)

            @plsc.parallel_loop(0, _CHUNK // 16, unroll=4)
            def grp_body(g_):
                wvec = w_b[bi, pl.ds(16 * g_, 16)]
                for j in range(16):
                    e = 16 * g_ + j
                    wb = lax.gather(
                        wvec, jnp.full((16, 1), j, jnp.int32),
                        lax.GatherDimensionNumbers(
                            offset_dims=(), collapsed_slice_dims=(0,),
                            start_index_map=(0,)),
                        slice_sizes=(1,),
                        mode=lax.GatherScatterMode.PROMISE_IN_BOUNDS)
                    for k in range(_D // 32):
                        vi = gbuf[si, e, pl.ds(16 * k, 16)]
                        a = plsc.bitcast(
                            lax.shift_left(vi, 16), jnp.float32)
                        b = plsc.bitcast(
                            jnp.bitwise_and(vi, jnp.int32(-65536)),
                            jnp.float32)
                        sbuf[si, e, pl.ds(32 * k, 16)] = a * wb
                        sbuf[si, e, pl.ds(32 * k + 16, 16)] = b * wb

            _fire_scatter(ci)
            return 0
        lax.fori_loop(0, _NCH, chunk_body, 0)

        for t in range(min(_NSB, _NCH)):
            _wait_scatter(_NCH - min(_NSB, _NCH) + t)

        plsc.subcore_barrier()

        pltpu.sync_copy(acc_sh.at[pl.ds(base, _RPT)],
                        out_hbm.at[c, pl.ds(base, _RPT)])
        if with_rs:
            pltpu.sync_copy(rs_sh.at[pl.ds(base, _RPT)],
                            rs_hbm.at[c, pl.ds(base, _RPT)])

    if with_rs:
        acc, rs = spmm(x, src3, dst3, w3)
        return acc, rs.reshape(_NC, _NPAD, 1)
    return spmm(x, src3, dst3, w3)


def _mlp_body(x_ref, p_ref, rs_ref, w_ref, b_ref, o_ref, *, relu):
    acc = p_ref[0] + p_ref[1]
    rs = rs_ref[0] + rs_ref[1]
    nacc = acc / jnp.maximum(rs, 1e-12)
    cat = jnp.concatenate([x_ref[...], nacc], axis=1)
    h = lax.dot_general(cat, w_ref[...], (((1,), (1,)), ((), ())),
                        preferred_element_type=jnp.float32) + b_ref[...]
    o_ref[...] = jnp.maximum(h, 0.0) if relu else h


_BLK = 1000


def _mlp_tc(x, part, rs, W, b, relu):
    body = functools.partial(_mlp_body, relu=relu)
    return pl.pallas_call(
        body,
        grid=(_N // _BLK,),
        in_specs=[
            pl.BlockSpec((_BLK, _D), lambda i: (i, 0)),
            pl.BlockSpec((_NC, _BLK, _D), lambda i: (0, i, 0)),
            pl.BlockSpec((_NC, _BLK, 1), lambda i: (0, i, 0)),
            pl.BlockSpec(W.shape, lambda i: (0, 0)),
            pl.BlockSpec((1, _D), lambda i: (0, 0)),
        ],
        out_specs=pl.BlockSpec((_BLK, _D), lambda i: (i, 0)),
        out_shape=jax.ShapeDtypeStruct((_N, _D), jnp.float32),
    )(x, part, rs, W, b)


def kernel(x, edge_index, edge_weight, W1_w, W1_b, W2_w, W2_b):
    dst = edge_index[0]
    src = edge_index[1]
    pad = _EPAD - _E
    src3 = jnp.concatenate([src, jnp.zeros((pad,), src.dtype)]).reshape(
        _NW, _NCH, _CHUNK)
    dst3 = jnp.concatenate([dst, jnp.zeros((pad,), dst.dtype)]).reshape(
        _NW, _NCH, _CHUNK)
    w3 = jnp.concatenate(
        [edge_weight, jnp.zeros((pad,), edge_weight.dtype)]).reshape(
        _NW, _NCH, _CHUNK)

    perm = jnp.array(_PERM, dtype=jnp.int32)

    def _pack(a):
        ab = jnp.take(a.astype(jnp.bfloat16), perm, axis=1)
        return lax.bitcast_convert_type(
            ab.reshape(_N, _D // 2, 2), jnp.int32)

    part1, rs1 = _spmm_sc(_pack(x), src3, dst3, w3, with_rs=True)
    h = _mlp_tc(x, part1, rs1, W1_w, W1_b.reshape(1, _D), relu=True)
    part2 = _spmm_sc(_pack(h), src3, dst3, w3, with_rs=False)
    return _mlp_tc(h, part2, rs1, W2_w, W2_b.reshape(1, _D), relu=False)
